# Initial kernel scaffold; baseline (speedup 1.0000x reference)
import jax
import jax.numpy as jnp
from jax.experimental import pallas as pl


def kernel(x, edge_index, W0, b0, W1, b1, W2, b2, W3, b3, W4, b4, W5, b5, W6, b6, W7, b7, W8, b8, W9, b9):
    raise NotImplementedError("write your pallas kernel here")


# speedup vs baseline: 12.0744x; 12.0744x over previous
import functools

import jax
import jax.numpy as jnp
from jax import lax
from jax.experimental import pallas as pl
from jax.experimental.pallas import tpu as pltpu
from jax.experimental.pallas import tpu_sc as plsc

N = 10000
E = 320000
E_TOT = E + N
NW = 32
C = 128
NCHUNK = -(-E_TOT // (NW * C))
E_PAD = NW * C * NCHUNK
N_ACC = 10112
ZROWS = N_ACC // 16
DIMS = [128, 256, 128, 64, 32, 16, 32, 64, 128, 256, 128]
NUM_LAYERS = 10
BLK = 128
GRID = -(-N // BLK)


def _sc_mesh():
    return plsc.VectorSubcoreMesh(core_axis_name="c", subcore_axis_name="s")


@functools.lru_cache(maxsize=None)
def _agg_call(w):
    @functools.partial(
        pl.kernel,
        mesh=_sc_mesh(),
        compiler_params=pltpu.CompilerParams(use_tc_tiling_on_sc=False),
        out_type=jax.ShapeDtypeStruct((2, N_ACC, w), jnp.float32),
        scratch_types=[
            pltpu.VMEM((NCHUNK, C), jnp.int32),
            pltpu.VMEM((NCHUNK, C), jnp.int32),
            pltpu.VMEM((C, w), jnp.float32),
            pltpu.VMEM_SHARED((N_ACC, w), jnp.float32),
            pltpu.SemaphoreType.DMA,
        ],
    )
    def agg(g_hbm, src_hbm, dst_hbm, zero_hbm, out_hbm, srcv, dstv, buf, acc, sem):
        ci = lax.axis_index("c")
        si = lax.axis_index("s")
        wid = si * 2 + ci
        pltpu.sync_copy(zero_hbm, acc.at[pl.ds(si * ZROWS, ZROWS)])
        pltpu.sync_copy(src_hbm.at[wid], srcv)
        pltpu.sync_copy(dst_hbm.at[wid], dstv)
        plsc.subcore_barrier()

        def body(j, carry):
            pltpu.async_copy(g_hbm.at[srcv.at[j]], buf, sem).wait()
            pltpu.sync_copy(buf, acc.at[dstv.at[j]], add=True)
            return carry

        lax.fori_loop(0, NCHUNK, body, 0)
        plsc.subcore_barrier()
        pltpu.sync_copy(acc.at[pl.ds(si * ZROWS, ZROWS)],
                        out_hbm.at[ci, pl.ds(si * ZROWS, ZROWS)])

    return agg


@functools.lru_cache(maxsize=None)
def _deg_call():
    @functools.partial(
        pl.kernel,
        mesh=_sc_mesh(),
        compiler_params=pltpu.CompilerParams(use_tc_tiling_on_sc=False),
        out_type=jax.ShapeDtypeStruct((2, N_ACC, 16), jnp.float32),
        scratch_types=[
            pltpu.VMEM((NCHUNK, C), jnp.int32),
            pltpu.VMEM((C, 16), jnp.float32),
            pltpu.VMEM_SHARED((N_ACC, 16), jnp.float32),
        ],
    )
    def deg(dst_hbm, ones_hbm, zero_hbm, out_hbm, dstv, buf, acc):
        ci = lax.axis_index("c")
        si = lax.axis_index("s")
        wid = si * 2 + ci
        pltpu.sync_copy(zero_hbm, acc.at[pl.ds(si * ZROWS, ZROWS)])
        pltpu.sync_copy(ones_hbm, buf)
        pltpu.sync_copy(dst_hbm.at[wid], dstv)
        plsc.subcore_barrier()

        def body(j, carry):
            pltpu.sync_copy(buf, acc.at[dstv.at[j]], add=True)
            return carry

        lax.fori_loop(0, NCHUNK, body, 0)
        plsc.subcore_barrier()
        pltpu.sync_copy(acc.at[pl.ds(si * ZROWS, ZROWS)],
                        out_hbm.at[ci, pl.ds(si * ZROWS, ZROWS)])

    return deg


def _stage_a(degp, x):
    def body(degp_ref, x_ref, dinv_ref, g_ref):
        deg = degp_ref[0, :, 0:1] + degp_ref[1, :, 0:1]
        dinv = jnp.where(deg > 0, lax.rsqrt(jnp.maximum(deg, 1e-12)), 0.0)
        dinv_ref[...] = dinv
        g_ref[...] = x_ref[...] * dinv

    return pl.pallas_call(
        body,
        grid=(GRID,),
        in_specs=[
            pl.BlockSpec((2, BLK, 16), lambda i: (0, i, 0)),
            pl.BlockSpec((BLK, DIMS[0]), lambda i: (i, 0)),
        ],
        out_specs=[
            pl.BlockSpec((BLK, 1), lambda i: (i, 0)),
            pl.BlockSpec((BLK, DIMS[0]), lambda i: (i, 0)),
        ],
        out_shape=[
            jax.ShapeDtypeStruct((N, 1), jnp.float32),
            jax.ShapeDtypeStruct((N, DIMS[0]), jnp.float32),
        ],
    )(degp, x)


def _stage_mid(pp, dinv, wa, b, wb, relu, scale_out):
    w_in = pp.shape[2]
    d_mid = wa.shape[1] if wa is not None else w_in
    d_out = wb.shape[1] if wb is not None else d_mid

    def body(*refs):
        it = iter(refs)
        pp_ref = next(it)
        dinv_ref = next(it)
        wa_ref = next(it) if wa is not None else None
        b_ref = next(it)
        wb_ref = next(it) if wb is not None else None
        o_ref = next(it)
        dinv_b = dinv_ref[...]
        t = (pp_ref[0] + pp_ref[1]) * dinv_b
        if wa_ref is not None:
            t = jnp.dot(t, wa_ref[...], preferred_element_type=jnp.float32)
        t = t + b_ref[...]
        if relu:
            t = jnp.maximum(t, 0.0)
        if wb_ref is not None:
            t = jnp.dot(t, wb_ref[...], preferred_element_type=jnp.float32)
        if scale_out:
            t = t * dinv_b
        o_ref[...] = t

    in_specs = [
        pl.BlockSpec((2, BLK, w_in), lambda i: (0, i, 0)),
        pl.BlockSpec((BLK, 1), lambda i: (i, 0)),
    ]
    args = [pp, dinv]
    if wa is not None:
        in_specs.append(pl.BlockSpec(wa.shape, lambda i: (0, 0)))
        args.append(wa)
    in_specs.append(pl.BlockSpec((1, d_mid), lambda i: (0, 0)))
    args.append(b.reshape(1, d_mid))
    if wb is not None:
        in_specs.append(pl.BlockSpec(wb.shape, lambda i: (0, 0)))
        args.append(wb)

    return pl.pallas_call(
        body,
        grid=(GRID,),
        in_specs=in_specs,
        out_specs=pl.BlockSpec((BLK, d_out), lambda i: (i, 0)),
        out_shape=jax.ShapeDtypeStruct((N, d_out), jnp.float32),
    )(*args)


def kernel(x, edge_index, W0, b0, W1, b1, W2, b2, W3, b3, W4, b4,
           W5, b5, W6, b6, W7, b7, W8, b8, W9, b9):
    Ws = [W0, W1, W2, W3, W4, W5, W6, W7, W8, W9]
    bs = [b0, b1, b2, b3, b4, b5, b6, b7, b8, b9]

    loop = jnp.arange(N, dtype=jnp.int32)
    src = jnp.concatenate([edge_index[0], loop,
                           jnp.zeros((E_PAD - E_TOT,), jnp.int32)])
    dst = jnp.concatenate([edge_index[1], loop,
                           jnp.full((E_PAD - E_TOT,), N, jnp.int32)])
    src3 = src.reshape(NW, NCHUNK, C)
    dst3 = dst.reshape(NW, NCHUNK, C)

    ones16 = jnp.zeros((C, 16), jnp.float32).at[:, 0].set(1.0)
    zeros = {w: jnp.zeros((ZROWS, w), jnp.float32) for w in (16, 32, 64, 128)}

    degp = _deg_call()(dst3, ones16, zeros[16])
    dinv, g = _stage_a(degp, x)

    agg_first = [DIMS[i] <= DIMS[i + 1] for i in range(NUM_LAYERS)]

    for i in range(NUM_LAYERS):
        pp = _agg_call(g.shape[1])(g, src3, dst3, zeros[g.shape[1]])
        wa = Ws[i] if agg_first[i] else None
        if i < NUM_LAYERS - 1:
            wb = None if agg_first[i + 1] else Ws[i + 1]
            g = _stage_mid(pp, dinv, wa, bs[i], wb, relu=True, scale_out=True)
        else:
            g = _stage_mid(pp, dinv, wa, bs[i], None, relu=False, scale_out=False)
    return g


# speedup vs baseline: 18.9235x; 1.5672x over previous
import functools

import jax
import jax.numpy as jnp
from jax import lax
from jax.experimental import pallas as pl
from jax.experimental.pallas import tpu as pltpu
from jax.experimental.pallas import tpu_sc as plsc

N = 10000
E = 320000
E_TOT = E + N
NW = 32
C = 96
_NC0 = -(-E_TOT // (NW * C))
NCHUNK = _NC0 + (_NC0 % 2)
E_PAD = NW * C * NCHUNK
N_ACC = 10112
ZROWS = N_ACC // 16
DIMS = [128, 256, 128, 64, 32, 16, 32, 64, 128, 256, 128]
NUM_LAYERS = 10
BLK = 128
GRID = -(-N // BLK)


def _sc_mesh():
    return plsc.VectorSubcoreMesh(core_axis_name="c", subcore_axis_name="s")


@functools.lru_cache(maxsize=None)
def _agg_call(w):
    @functools.partial(
        pl.kernel,
        mesh=_sc_mesh(),
        compiler_params=pltpu.CompilerParams(use_tc_tiling_on_sc=False),
        out_type=jax.ShapeDtypeStruct((2, N_ACC, w), jnp.float32),
        scratch_types=[
            pltpu.VMEM((NCHUNK, C), jnp.int32),
            pltpu.VMEM((NCHUNK, C), jnp.int32),
            pltpu.VMEM((C, w), jnp.float32),
            pltpu.VMEM((C, w), jnp.float32),
            pltpu.VMEM_SHARED((N_ACC, w), jnp.float32),
            pltpu.SemaphoreType.DMA,
            pltpu.SemaphoreType.DMA,
        ],
    )
    def agg(g_hbm, src_hbm, dst_hbm, zero_hbm, out_hbm,
            srcv, dstv, bufa, bufb, acc, sema, semb):
        ci = lax.axis_index("c")
        si = lax.axis_index("s")
        wid = si * 2 + ci
        pltpu.sync_copy(zero_hbm, acc.at[pl.ds(si * ZROWS, ZROWS)])
        pltpu.sync_copy(src_hbm.at[wid], srcv)
        pltpu.sync_copy(dst_hbm.at[wid], dstv)
        plsc.subcore_barrier()

        pltpu.async_copy(g_hbm.at[srcv.at[0]], bufa, sema)

        def body(i, carry):
            j0 = 2 * i
            pltpu.async_copy(g_hbm.at[srcv.at[j0 + 1]], bufb, semb)
            pltpu.make_async_copy(g_hbm.at[srcv.at[j0]], bufa, sema).wait()
            pltpu.sync_copy(bufa, acc.at[dstv.at[j0]], add=True)
            jn = jnp.minimum(j0 + 2, NCHUNK - 1)
            pltpu.async_copy(g_hbm.at[srcv.at[jn]], bufa, sema)
            pltpu.make_async_copy(g_hbm.at[srcv.at[j0 + 1]], bufb, semb).wait()
            pltpu.sync_copy(bufb, acc.at[dstv.at[j0 + 1]], add=True)
            return carry

        lax.fori_loop(0, NCHUNK // 2, body, 0)
        pltpu.make_async_copy(g_hbm.at[srcv.at[0]], bufa, sema).wait()
        plsc.subcore_barrier()
        pltpu.sync_copy(acc.at[pl.ds(si * ZROWS, ZROWS)],
                        out_hbm.at[ci, pl.ds(si * ZROWS, ZROWS)])

    return agg


@functools.lru_cache(maxsize=None)
def _deg_call():
    @functools.partial(
        pl.kernel,
        mesh=_sc_mesh(),
        compiler_params=pltpu.CompilerParams(use_tc_tiling_on_sc=False),
        out_type=jax.ShapeDtypeStruct((2, N_ACC, 16), jnp.float32),
        scratch_types=[
            pltpu.VMEM((NCHUNK, C), jnp.int32),
            pltpu.VMEM((C, 16), jnp.float32),
            pltpu.VMEM_SHARED((N_ACC, 16), jnp.float32),
        ],
    )
    def deg(dst_hbm, ones_hbm, zero_hbm, out_hbm, dstv, buf, acc):
        ci = lax.axis_index("c")
        si = lax.axis_index("s")
        wid = si * 2 + ci
        pltpu.sync_copy(zero_hbm, acc.at[pl.ds(si * ZROWS, ZROWS)])
        pltpu.sync_copy(ones_hbm, buf)
        pltpu.sync_copy(dst_hbm.at[wid], dstv)
        plsc.subcore_barrier()

        def body(j, carry):
            pltpu.sync_copy(buf, acc.at[dstv.at[j]], add=True)
            return carry

        lax.fori_loop(0, NCHUNK, body, 0)
        plsc.subcore_barrier()
        pltpu.sync_copy(acc.at[pl.ds(si * ZROWS, ZROWS)],
                        out_hbm.at[ci, pl.ds(si * ZROWS, ZROWS)])

    return deg


def _stage_a(degp, x):
    def body(degp_ref, x_ref, dinv_ref, g_ref):
        deg = degp_ref[0, :, 0:1] + degp_ref[1, :, 0:1]
        dinv = jnp.where(deg > 0, lax.rsqrt(jnp.maximum(deg, 1e-12)), 0.0)
        dinv_ref[...] = dinv
        g_ref[...] = x_ref[...] * dinv

    return pl.pallas_call(
        body,
        grid=(GRID,),
        in_specs=[
            pl.BlockSpec((2, BLK, 16), lambda i: (0, i, 0)),
            pl.BlockSpec((BLK, DIMS[0]), lambda i: (i, 0)),
        ],
        out_specs=[
            pl.BlockSpec((BLK, 1), lambda i: (i, 0)),
            pl.BlockSpec((BLK, DIMS[0]), lambda i: (i, 0)),
        ],
        out_shape=[
            jax.ShapeDtypeStruct((N, 1), jnp.float32),
            jax.ShapeDtypeStruct((N, DIMS[0]), jnp.float32),
        ],
    )(degp, x)


def _stage_mid(pp, dinv, wa, b, wb, relu, scale_out):
    w_in = pp.shape[2]
    d_mid = wa.shape[1] if wa is not None else w_in
    d_out = wb.shape[1] if wb is not None else d_mid

    def body(*refs):
        it = iter(refs)
        pp_ref = next(it)
        dinv_ref = next(it)
        wa_ref = next(it) if wa is not None else None
        b_ref = next(it)
        wb_ref = next(it) if wb is not None else None
        o_ref = next(it)
        dinv_b = dinv_ref[...]
        t = (pp_ref[0] + pp_ref[1]) * dinv_b
        if wa_ref is not None:
            t = jnp.dot(t, wa_ref[...], preferred_element_type=jnp.float32)
        t = t + b_ref[...]
        if relu:
            t = jnp.maximum(t, 0.0)
        if wb_ref is not None:
            t = jnp.dot(t, wb_ref[...], preferred_element_type=jnp.float32)
        if scale_out:
            t = t * dinv_b
        o_ref[...] = t

    in_specs = [
        pl.BlockSpec((2, BLK, w_in), lambda i: (0, i, 0)),
        pl.BlockSpec((BLK, 1), lambda i: (i, 0)),
    ]
    args = [pp, dinv]
    if wa is not None:
        in_specs.append(pl.BlockSpec(wa.shape, lambda i: (0, 0)))
        args.append(wa)
    in_specs.append(pl.BlockSpec((1, d_mid), lambda i: (0, 0)))
    args.append(b.reshape(1, d_mid))
    if wb is not None:
        in_specs.append(pl.BlockSpec(wb.shape, lambda i: (0, 0)))
        args.append(wb)

    return pl.pallas_call(
        body,
        grid=(GRID,),
        in_specs=in_specs,
        out_specs=pl.BlockSpec((BLK, d_out), lambda i: (i, 0)),
        out_shape=jax.ShapeDtypeStruct((N, d_out), jnp.float32),
    )(*args)


def kernel(x, edge_index, W0, b0, W1, b1, W2, b2, W3, b3, W4, b4,
           W5, b5, W6, b6, W7, b7, W8, b8, W9, b9):
    Ws = [W0, W1, W2, W3, W4, W5, W6, W7, W8, W9]
    bs = [b0, b1, b2, b3, b4, b5, b6, b7, b8, b9]

    loop = jnp.arange(N, dtype=jnp.int32)
    padn = E_PAD - E_TOT
    pad_iota = jnp.arange(padn, dtype=jnp.int32)
    src = jnp.concatenate([edge_index[0], loop, pad_iota % N])
    dst = jnp.concatenate([edge_index[1], loop, N + pad_iota % (N_ACC - N)])
    src3 = src.reshape(NW, NCHUNK, C)
    dst3 = dst.reshape(NW, NCHUNK, C)

    ones16 = jnp.zeros((C, 16), jnp.float32).at[:, 0].set(1.0)
    zeros = {w: jnp.zeros((ZROWS, w), jnp.float32) for w in (16, 32, 64, 128)}

    degp = _deg_call()(dst3, ones16, zeros[16])
    dinv, g = _stage_a(degp, x)

    agg_first = [DIMS[i] <= DIMS[i + 1] for i in range(NUM_LAYERS)]

    for i in range(NUM_LAYERS):
        pp = _agg_call(g.shape[1])(g, src3, dst3, zeros[g.shape[1]])
        wa = Ws[i] if agg_first[i] else None
        if i < NUM_LAYERS - 1:
            wb = None if agg_first[i + 1] else Ws[i + 1]
            g = _stage_mid(pp, dinv, wa, bs[i], wb, relu=True, scale_out=True)
        else:
            g = _stage_mid(pp, dinv, wa, bs[i], None, relu=False, scale_out=False)
    return g


# speedup vs baseline: 25.7658x; 1.3616x over previous
import functools

import jax
import jax.numpy as jnp
from jax import lax
from jax.experimental import pallas as pl
from jax.experimental.pallas import tpu as pltpu
from jax.experimental.pallas import tpu_sc as plsc

N = 10000
E = 320000
E_TOT = E + N
NW = 32
C = 96
_NC0 = -(-E_TOT // (NW * C))
NCHUNK = _NC0 + (_NC0 % 2)
E_PAD = NW * C * NCHUNK
N_ACC = 10112
ZROWS = N_ACC // 16
DIMS = [128, 256, 128, 64, 32, 16, 32, 64, 128, 256, 128]
NUM_LAYERS = 10
BLK = 1024
GRID = -(-N // BLK)


def _sc_mesh():
    return plsc.VectorSubcoreMesh(core_axis_name="c", subcore_axis_name="s")


@functools.lru_cache(maxsize=None)
def _agg_call(w):
    @functools.partial(
        pl.kernel,
        mesh=_sc_mesh(),
        compiler_params=pltpu.CompilerParams(use_tc_tiling_on_sc=False),
        out_type=jax.ShapeDtypeStruct((2, N_ACC, w), jnp.float32),
        scratch_types=[
            pltpu.VMEM((NCHUNK, C), jnp.int32),
            pltpu.VMEM((NCHUNK, C), jnp.int32),
            pltpu.VMEM((C, w), jnp.float32),
            pltpu.VMEM((C, w), jnp.float32),
            pltpu.VMEM_SHARED((N_ACC, w), jnp.float32),
            pltpu.SemaphoreType.DMA,
            pltpu.SemaphoreType.DMA,
        ],
    )
    def agg(g_hbm, src_hbm, dst_hbm, zero_hbm, out_hbm,
            srcv, dstv, bufa, bufb, acc, sema, semb):
        ci = lax.axis_index("c")
        si = lax.axis_index("s")
        wid = si * 2 + ci
        pltpu.sync_copy(zero_hbm, acc.at[pl.ds(si * ZROWS, ZROWS)])
        pltpu.sync_copy(src_hbm.at[wid], srcv)
        pltpu.sync_copy(dst_hbm.at[wid], dstv)
        plsc.subcore_barrier()

        pltpu.async_copy(g_hbm.at[srcv.at[0]], bufa, sema)

        def body(i, carry):
            j0 = 2 * i
            pltpu.async_copy(g_hbm.at[srcv.at[j0 + 1]], bufb, semb)
            pltpu.make_async_copy(g_hbm.at[srcv.at[j0]], bufa, sema).wait()
            pltpu.sync_copy(bufa, acc.at[dstv.at[j0]], add=True)
            jn = jnp.minimum(j0 + 2, NCHUNK - 1)
            pltpu.async_copy(g_hbm.at[srcv.at[jn]], bufa, sema)
            pltpu.make_async_copy(g_hbm.at[srcv.at[j0 + 1]], bufb, semb).wait()
            pltpu.sync_copy(bufb, acc.at[dstv.at[j0 + 1]], add=True)
            return carry

        lax.fori_loop(0, NCHUNK // 2, body, 0)
        pltpu.make_async_copy(g_hbm.at[srcv.at[0]], bufa, sema).wait()
        plsc.subcore_barrier()
        pltpu.sync_copy(acc.at[pl.ds(si * ZROWS, ZROWS)],
                        out_hbm.at[ci, pl.ds(si * ZROWS, ZROWS)])

    return agg


@functools.lru_cache(maxsize=None)
def _deg_call():
    @functools.partial(
        pl.kernel,
        mesh=_sc_mesh(),
        compiler_params=pltpu.CompilerParams(use_tc_tiling_on_sc=False),
        out_type=jax.ShapeDtypeStruct((2, N_ACC, 16), jnp.float32),
        scratch_types=[
            pltpu.VMEM((NCHUNK, C), jnp.int32),
            pltpu.VMEM((C, 16), jnp.float32),
            pltpu.VMEM_SHARED((N_ACC, 16), jnp.float32),
        ],
    )
    def deg(dst_hbm, ones_hbm, zero_hbm, out_hbm, dstv, buf, acc):
        ci = lax.axis_index("c")
        si = lax.axis_index("s")
        wid = si * 2 + ci
        pltpu.sync_copy(zero_hbm, acc.at[pl.ds(si * ZROWS, ZROWS)])
        pltpu.sync_copy(ones_hbm, buf)
        pltpu.sync_copy(dst_hbm.at[wid], dstv)
        plsc.subcore_barrier()

        def body(j, carry):
            pltpu.sync_copy(buf, acc.at[dstv.at[j]], add=True)
            return carry

        lax.fori_loop(0, NCHUNK, body, 0)
        plsc.subcore_barrier()
        pltpu.sync_copy(acc.at[pl.ds(si * ZROWS, ZROWS)],
                        out_hbm.at[ci, pl.ds(si * ZROWS, ZROWS)])

    return deg


def _stage_a(degp, x):
    def body(degp_ref, x_ref, dinv_ref, g_ref):
        deg = degp_ref[0, :, 0:1] + degp_ref[1, :, 0:1]
        dinv = jnp.where(deg > 0, lax.rsqrt(jnp.maximum(deg, 1e-12)), 0.0)
        dinv_ref[...] = dinv
        g_ref[...] = x_ref[...] * dinv

    return pl.pallas_call(
        body,
        grid=(GRID,),
        in_specs=[
            pl.BlockSpec((2, BLK, 16), lambda i: (0, i, 0)),
            pl.BlockSpec((BLK, DIMS[0]), lambda i: (i, 0)),
        ],
        out_specs=[
            pl.BlockSpec((BLK, 1), lambda i: (i, 0)),
            pl.BlockSpec((BLK, DIMS[0]), lambda i: (i, 0)),
        ],
        out_shape=[
            jax.ShapeDtypeStruct((N, 1), jnp.float32),
            jax.ShapeDtypeStruct((N, DIMS[0]), jnp.float32),
        ],
    )(degp, x)


def _stage_mid(pp, dinv, wa, b, wb, relu, scale_out):
    w_in = pp.shape[2]
    d_mid = wa.shape[1] if wa is not None else w_in
    d_out = wb.shape[1] if wb is not None else d_mid

    def body(*refs):
        it = iter(refs)
        pp_ref = next(it)
        dinv_ref = next(it)
        wa_ref = next(it) if wa is not None else None
        b_ref = next(it)
        wb_ref = next(it) if wb is not None else None
        o_ref = next(it)
        dinv_b = dinv_ref[...]
        t = (pp_ref[0] + pp_ref[1]) * dinv_b
        if wa_ref is not None:
            t = jnp.dot(t, wa_ref[...], preferred_element_type=jnp.float32)
        t = t + b_ref[...][None, :]
        if relu:
            t = jnp.maximum(t, 0.0)
        if wb_ref is not None:
            t = jnp.dot(t, wb_ref[...], preferred_element_type=jnp.float32)
        if scale_out:
            t = t * dinv_b
        o_ref[...] = t

    in_specs = [
        pl.BlockSpec((2, BLK, w_in), lambda i: (0, i, 0)),
        pl.BlockSpec((BLK, 1), lambda i: (i, 0)),
    ]
    args = [pp, dinv]
    if wa is not None:
        in_specs.append(pl.BlockSpec(wa.shape, lambda i: (0, 0)))
        args.append(wa)
    in_specs.append(pl.BlockSpec((d_mid,), lambda i: (0,)))
    args.append(b)
    if wb is not None:
        in_specs.append(pl.BlockSpec(wb.shape, lambda i: (0, 0)))
        args.append(wb)

    return pl.pallas_call(
        body,
        grid=(GRID,),
        in_specs=in_specs,
        out_specs=pl.BlockSpec((BLK, d_out), lambda i: (i, 0)),
        out_shape=jax.ShapeDtypeStruct((N, d_out), jnp.float32),
    )(*args)


def kernel(x, edge_index, W0, b0, W1, b1, W2, b2, W3, b3, W4, b4,
           W5, b5, W6, b6, W7, b7, W8, b8, W9, b9):
    Ws = [W0, W1, W2, W3, W4, W5, W6, W7, W8, W9]
    bs = [b0, b1, b2, b3, b4, b5, b6, b7, b8, b9]

    loop = jnp.arange(N, dtype=jnp.int32)
    padn = E_PAD - E_TOT
    pad_iota = jnp.arange(padn, dtype=jnp.int32)
    src = jnp.concatenate([edge_index[0], loop, pad_iota % N])
    dst = jnp.concatenate([edge_index[1], loop, N + pad_iota % (N_ACC - N)])
    src3 = src.reshape(NW, NCHUNK, C)
    dst3 = dst.reshape(NW, NCHUNK, C)

    ones16 = jnp.zeros((C, 16), jnp.float32).at[:, 0].set(1.0)
    zeros = {w: jnp.zeros((ZROWS, w), jnp.float32) for w in (16, 32, 64, 128)}

    degp = _deg_call()(dst3, ones16, zeros[16])
    dinv, g = _stage_a(degp, x)

    agg_first = [DIMS[i] <= DIMS[i + 1] for i in range(NUM_LAYERS)]

    for i in range(NUM_LAYERS):
        pp = _agg_call(g.shape[1])(g, src3, dst3, zeros[g.shape[1]])
        wa = Ws[i] if agg_first[i] else None
        if i < NUM_LAYERS - 1:
            wb = None if agg_first[i + 1] else Ws[i + 1]
            g = _stage_mid(pp, dinv, wa, bs[i], wb, relu=True, scale_out=True)
        else:
            g = _stage_mid(pp, dinv, wa, bs[i], None, relu=False, scale_out=False)
    return g


# speedup vs baseline: 27.5398x; 1.0689x over previous
import functools

import jax
import jax.numpy as jnp
from jax import lax
from jax.experimental import pallas as pl
from jax.experimental.pallas import tpu as pltpu
from jax.experimental.pallas import tpu_sc as plsc

N = 10000
E = 320000
E_TOT = E + N
NW = 32
C = 56
NBUF = 4
_NC0 = -(-E_TOT // (NW * C))
NCHUNK = _NC0 + (-_NC0 % NBUF)
E_PAD = NW * C * NCHUNK
N_ACC = 10112
ZROWS = N_ACC // 16
DIMS = [128, 256, 128, 64, 32, 16, 32, 64, 128, 256, 128]
NUM_LAYERS = 10
BLK = 1024
GRID = -(-N // BLK)


def _sc_mesh():
    return plsc.VectorSubcoreMesh(core_axis_name="c", subcore_axis_name="s")


@functools.lru_cache(maxsize=None)
def _agg_call(w):
    @functools.partial(
        pl.kernel,
        mesh=_sc_mesh(),
        compiler_params=pltpu.CompilerParams(use_tc_tiling_on_sc=False),
        out_type=jax.ShapeDtypeStruct((2, N_ACC, w), jnp.float32),
        scratch_types=[
            pltpu.VMEM((NCHUNK, C), jnp.int32),
            pltpu.VMEM((NCHUNK, C), jnp.int32),
        ] + [pltpu.VMEM((C, w), jnp.float32) for _ in range(NBUF)] + [
            pltpu.VMEM_SHARED((N_ACC, w), jnp.float32),
        ] + [pltpu.SemaphoreType.DMA] * (2 * NBUF),
    )
    def agg(g_hbm, src_hbm, dst_hbm, zero_hbm, out_hbm,
            srcv, dstv, b0, b1, b2, b3, acc, g0, g1, g2, g3, s0, s1, s2, s3):
        bufs = (b0, b1, b2, b3)
        semg = (g0, g1, g2, g3)
        sems = (s0, s1, s2, s3)
        ci = lax.axis_index("c")
        si = lax.axis_index("s")
        wid = si * 2 + ci
        pltpu.sync_copy(zero_hbm, acc.at[pl.ds(si * ZROWS, ZROWS)])
        pltpu.sync_copy(src_hbm.at[wid], srcv)
        pltpu.sync_copy(dst_hbm.at[wid], dstv)
        plsc.subcore_barrier()

        def g_fire(j, s):
            pltpu.async_copy(g_hbm.at[srcv.at[j]], bufs[s], semg[s])

        def g_wait(s):
            pltpu.make_async_copy(g_hbm.at[srcv.at[0]], bufs[s], semg[s]).wait()

        def s_fire(j, s):
            pltpu.async_copy(bufs[s], acc.at[dstv.at[j]], sems[s], add=True)

        def s_wait(s):
            pltpu.make_async_copy(bufs[s], acc.at[dstv.at[0]], sems[s]).wait()

        g_fire(0, 0)
        g_fire(1, 1)
        g_fire(2, 2)
        g_wait(0)
        s_fire(0, 0)
        g_fire(3, 3)
        g_wait(1)
        s_fire(1, 1)
        s_wait(0)
        g_fire(4, 0)
        g_wait(2)
        s_fire(2, 2)
        s_wait(1)
        g_fire(5, 1)
        g_wait(3)
        s_fire(3, 3)

        def body(g, carry):
            j0 = 4 * g
            for s in range(NBUF):
                j = j0 + s
                s2 = (s + 2) % NBUF
                s_wait(s2)
                g_fire(jnp.minimum(j + 2, NCHUNK - 1), s2)
                g_wait(s)
                s_fire(j, s)
            return carry

        lax.fori_loop(1, NCHUNK // NBUF, body, 0)
        g_wait(0)
        g_wait(1)
        s_wait(2)
        s_wait(3)
        plsc.subcore_barrier()
        pltpu.sync_copy(acc.at[pl.ds(si * ZROWS, ZROWS)],
                        out_hbm.at[ci, pl.ds(si * ZROWS, ZROWS)])

    return agg


@functools.lru_cache(maxsize=None)
def _deg_call():
    @functools.partial(
        pl.kernel,
        mesh=_sc_mesh(),
        compiler_params=pltpu.CompilerParams(use_tc_tiling_on_sc=False),
        out_type=jax.ShapeDtypeStruct((2, N_ACC, 16), jnp.float32),
        scratch_types=[
            pltpu.VMEM((NCHUNK, C), jnp.int32),
            pltpu.VMEM((C, 16), jnp.float32),
            pltpu.VMEM_SHARED((N_ACC, 16), jnp.float32),
        ] + [pltpu.SemaphoreType.DMA] * NBUF,
    )
    def deg(dst_hbm, ones_hbm, zero_hbm, out_hbm, dstv, buf, acc, s0, s1, s2, s3):
        sems = (s0, s1, s2, s3)
        ci = lax.axis_index("c")
        si = lax.axis_index("s")
        wid = si * 2 + ci
        pltpu.sync_copy(zero_hbm, acc.at[pl.ds(si * ZROWS, ZROWS)])
        pltpu.sync_copy(ones_hbm, buf)
        pltpu.sync_copy(dst_hbm.at[wid], dstv)
        plsc.subcore_barrier()

        def s_fire(j, s):
            pltpu.async_copy(buf, acc.at[dstv.at[j]], sems[s], add=True)

        def s_wait(s):
            pltpu.make_async_copy(buf, acc.at[dstv.at[0]], sems[s]).wait()

        for s in range(NBUF):
            s_fire(s, s)

        def body(g, carry):
            j0 = NBUF * g
            for s in range(NBUF):
                s_wait(s)
                s_fire(j0 + s, s)
            return carry

        lax.fori_loop(1, NCHUNK // NBUF, body, 0)
        for s in range(NBUF):
            s_wait(s)
        plsc.subcore_barrier()
        pltpu.sync_copy(acc.at[pl.ds(si * ZROWS, ZROWS)],
                        out_hbm.at[ci, pl.ds(si * ZROWS, ZROWS)])

    return deg


def _stage_a(degp, x):
    def body(degp_ref, x_ref, dinv_ref, g_ref):
        deg = degp_ref[0, :, 0:1] + degp_ref[1, :, 0:1]
        dinv = jnp.where(deg > 0, lax.rsqrt(jnp.maximum(deg, 1e-12)), 0.0)
        dinv_ref[...] = dinv
        g_ref[...] = x_ref[...] * dinv

    return pl.pallas_call(
        body,
        grid=(GRID,),
        in_specs=[
            pl.BlockSpec((2, BLK, 16), lambda i: (0, i, 0)),
            pl.BlockSpec((BLK, DIMS[0]), lambda i: (i, 0)),
        ],
        out_specs=[
            pl.BlockSpec((BLK, 1), lambda i: (i, 0)),
            pl.BlockSpec((BLK, DIMS[0]), lambda i: (i, 0)),
        ],
        out_shape=[
            jax.ShapeDtypeStruct((N, 1), jnp.float32),
            jax.ShapeDtypeStruct((N, DIMS[0]), jnp.float32),
        ],
    )(degp, x)


def _stage_mid(pp, dinv, wa, b, wb, relu, scale_out):
    w_in = pp.shape[2]
    d_mid = wa.shape[1] if wa is not None else w_in
    d_out = wb.shape[1] if wb is not None else d_mid

    def body(*refs):
        it = iter(refs)
        pp_ref = next(it)
        dinv_ref = next(it)
        wa_ref = next(it) if wa is not None else None
        b_ref = next(it)
        wb_ref = next(it) if wb is not None else None
        o_ref = next(it)
        dinv_b = dinv_ref[...]
        t = (pp_ref[0] + pp_ref[1]) * dinv_b
        if wa_ref is not None:
            t = jnp.dot(t, wa_ref[...], preferred_element_type=jnp.float32)
        t = t + b_ref[...][None, :]
        if relu:
            t = jnp.maximum(t, 0.0)
        if wb_ref is not None:
            t = jnp.dot(t, wb_ref[...], preferred_element_type=jnp.float32)
        if scale_out:
            t = t * dinv_b
        o_ref[...] = t

    in_specs = [
        pl.BlockSpec((2, BLK, w_in), lambda i: (0, i, 0)),
        pl.BlockSpec((BLK, 1), lambda i: (i, 0)),
    ]
    args = [pp, dinv]
    if wa is not None:
        in_specs.append(pl.BlockSpec(wa.shape, lambda i: (0, 0)))
        args.append(wa)
    in_specs.append(pl.BlockSpec((d_mid,), lambda i: (0,)))
    args.append(b)
    if wb is not None:
        in_specs.append(pl.BlockSpec(wb.shape, lambda i: (0, 0)))
        args.append(wb)

    return pl.pallas_call(
        body,
        grid=(GRID,),
        in_specs=in_specs,
        out_specs=pl.BlockSpec((BLK, d_out), lambda i: (i, 0)),
        out_shape=jax.ShapeDtypeStruct((N, d_out), jnp.float32),
    )(*args)


def kernel(x, edge_index, W0, b0, W1, b1, W2, b2, W3, b3, W4, b4,
           W5, b5, W6, b6, W7, b7, W8, b8, W9, b9):
    Ws = [W0, W1, W2, W3, W4, W5, W6, W7, W8, W9]
    bs = [b0, b1, b2, b3, b4, b5, b6, b7, b8, b9]

    loop = jnp.arange(N, dtype=jnp.int32)
    padn = E_PAD - E_TOT
    pad_iota = jnp.arange(padn, dtype=jnp.int32)
    src = jnp.concatenate([edge_index[0], loop, pad_iota % N])
    dst = jnp.concatenate([edge_index[1], loop, N + pad_iota % (N_ACC - N)])
    src3 = src.reshape(NW, NCHUNK, C)
    dst3 = dst.reshape(NW, NCHUNK, C)

    ones16 = jnp.zeros((C, 16), jnp.float32).at[:, 0].set(1.0)
    zeros = {w: jnp.zeros((ZROWS, w), jnp.float32) for w in (16, 32, 64, 128)}

    degp = _deg_call()(dst3, ones16, zeros[16])
    dinv, g = _stage_a(degp, x)

    agg_first = [DIMS[i] <= DIMS[i + 1] for i in range(NUM_LAYERS)]

    for i in range(NUM_LAYERS):
        pp = _agg_call(g.shape[1])(g, src3, dst3, zeros[g.shape[1]])
        wa = Ws[i] if agg_first[i] else None
        if i < NUM_LAYERS - 1:
            wb = None if agg_first[i + 1] else Ws[i + 1]
            g = _stage_mid(pp, dinv, wa, bs[i], wb, relu=True, scale_out=True)
        else:
            g = _stage_mid(pp, dinv, wa, bs[i], None, relu=False, scale_out=False)
    return g


# speedup vs baseline: 30.0185x; 1.0900x over previous
import functools

import jax
import jax.numpy as jnp
from jax import lax
from jax.experimental import pallas as pl
from jax.experimental.pallas import tpu as pltpu
from jax.experimental.pallas import tpu_sc as plsc

N = 10000
E = 320000
E_TOT = E + N
NW = 32
NBUF = 4


def _geom(w):
    c = 56 if w == 128 else 128
    nc0 = -(-E_TOT // (NW * c))
    nchunk = nc0 + (-nc0 % NBUF)
    return c, nchunk, NW * c * nchunk
N_ACC = 10112
ZROWS = N_ACC // 16
DIMS = [128, 256, 128, 64, 32, 16, 32, 64, 128, 256, 128]
NUM_LAYERS = 10
BLK = 1024
GRID = -(-N // BLK)


def _sc_mesh():
    return plsc.VectorSubcoreMesh(core_axis_name="c", subcore_axis_name="s")


@functools.lru_cache(maxsize=None)
def _agg_call(w):
    C, NCHUNK, _ = _geom(w)

    @functools.partial(
        pl.kernel,
        mesh=_sc_mesh(),
        compiler_params=pltpu.CompilerParams(use_tc_tiling_on_sc=False),
        out_type=jax.ShapeDtypeStruct((2, N_ACC, w), jnp.float32),
        scratch_types=[
            pltpu.VMEM((NCHUNK, C), jnp.int32),
            pltpu.VMEM((NCHUNK, C), jnp.int32),
        ] + [pltpu.VMEM((C, w), jnp.float32) for _ in range(NBUF)] + [
            pltpu.VMEM_SHARED((N_ACC, w), jnp.float32),
        ] + [pltpu.SemaphoreType.DMA] * (2 * NBUF),
    )
    def agg(g_hbm, src_hbm, dst_hbm, zero_hbm, out_hbm,
            srcv, dstv, b0, b1, b2, b3, acc, g0, g1, g2, g3, s0, s1, s2, s3):
        bufs = (b0, b1, b2, b3)
        semg = (g0, g1, g2, g3)
        sems = (s0, s1, s2, s3)
        ci = lax.axis_index("c")
        si = lax.axis_index("s")
        wid = si * 2 + ci
        pltpu.sync_copy(zero_hbm, acc.at[pl.ds(si * ZROWS, ZROWS)])
        pltpu.sync_copy(src_hbm.at[wid], srcv)
        pltpu.sync_copy(dst_hbm.at[wid], dstv)
        plsc.subcore_barrier()

        def g_fire(j, s):
            pltpu.async_copy(g_hbm.at[srcv.at[j]], bufs[s], semg[s])

        def g_wait(s):
            pltpu.make_async_copy(g_hbm.at[srcv.at[0]], bufs[s], semg[s]).wait()

        def s_fire(j, s):
            pltpu.async_copy(bufs[s], acc.at[dstv.at[j]], sems[s], add=True)

        def s_wait(s):
            pltpu.make_async_copy(bufs[s], acc.at[dstv.at[0]], sems[s]).wait()

        g_fire(0, 0)
        g_fire(1, 1)
        g_fire(2, 2)
        g_wait(0)
        s_fire(0, 0)
        g_fire(3, 3)
        g_wait(1)
        s_fire(1, 1)
        s_wait(0)
        g_fire(4, 0)
        g_wait(2)
        s_fire(2, 2)
        s_wait(1)
        g_fire(5, 1)
        g_wait(3)
        s_fire(3, 3)

        def body(g, carry):
            j0 = 4 * g
            for s in range(NBUF):
                j = j0 + s
                s2 = (s + 2) % NBUF
                s_wait(s2)
                g_fire(jnp.minimum(j + 2, NCHUNK - 1), s2)
                g_wait(s)
                s_fire(j, s)
            return carry

        lax.fori_loop(1, NCHUNK // NBUF, body, 0)
        g_wait(0)
        g_wait(1)
        s_wait(2)
        s_wait(3)
        plsc.subcore_barrier()
        pltpu.sync_copy(acc.at[pl.ds(si * ZROWS, ZROWS)],
                        out_hbm.at[ci, pl.ds(si * ZROWS, ZROWS)])

    return agg


@functools.lru_cache(maxsize=None)
def _deg_call():
    C, NCHUNK, _ = _geom(16)

    @functools.partial(
        pl.kernel,
        mesh=_sc_mesh(),
        compiler_params=pltpu.CompilerParams(use_tc_tiling_on_sc=False),
        out_type=jax.ShapeDtypeStruct((2, N_ACC, 16), jnp.float32),
        scratch_types=[
            pltpu.VMEM((NCHUNK, C), jnp.int32),
            pltpu.VMEM((C, 16), jnp.float32),
            pltpu.VMEM_SHARED((N_ACC, 16), jnp.float32),
        ] + [pltpu.SemaphoreType.DMA] * NBUF,
    )
    def deg(dst_hbm, ones_hbm, zero_hbm, out_hbm, dstv, buf, acc, s0, s1, s2, s3):
        sems = (s0, s1, s2, s3)
        ci = lax.axis_index("c")
        si = lax.axis_index("s")
        wid = si * 2 + ci
        pltpu.sync_copy(zero_hbm, acc.at[pl.ds(si * ZROWS, ZROWS)])
        pltpu.sync_copy(ones_hbm, buf)
        pltpu.sync_copy(dst_hbm.at[wid], dstv)
        plsc.subcore_barrier()

        def s_fire(j, s):
            pltpu.async_copy(buf, acc.at[dstv.at[j]], sems[s], add=True)

        def s_wait(s):
            pltpu.make_async_copy(buf, acc.at[dstv.at[0]], sems[s]).wait()

        for s in range(NBUF):
            s_fire(s, s)

        def body(g, carry):
            j0 = NBUF * g
            for s in range(NBUF):
                s_wait(s)
                s_fire(j0 + s, s)
            return carry

        lax.fori_loop(1, NCHUNK // NBUF, body, 0)
        for s in range(NBUF):
            s_wait(s)
        plsc.subcore_barrier()
        pltpu.sync_copy(acc.at[pl.ds(si * ZROWS, ZROWS)],
                        out_hbm.at[ci, pl.ds(si * ZROWS, ZROWS)])

    return deg


def _stage_a(degp, x):
    def body(degp_ref, x_ref, dinv_ref, g_ref):
        deg = degp_ref[0, :, 0:1] + degp_ref[1, :, 0:1]
        dinv = jnp.where(deg > 0, lax.rsqrt(jnp.maximum(deg, 1e-12)), 0.0)
        dinv_ref[...] = dinv
        g_ref[...] = x_ref[...] * dinv

    return pl.pallas_call(
        body,
        grid=(GRID,),
        in_specs=[
            pl.BlockSpec((2, BLK, 16), lambda i: (0, i, 0)),
            pl.BlockSpec((BLK, DIMS[0]), lambda i: (i, 0)),
        ],
        out_specs=[
            pl.BlockSpec((BLK, 1), lambda i: (i, 0)),
            pl.BlockSpec((BLK, DIMS[0]), lambda i: (i, 0)),
        ],
        out_shape=[
            jax.ShapeDtypeStruct((N, 1), jnp.float32),
            jax.ShapeDtypeStruct((N, DIMS[0]), jnp.float32),
        ],
    )(degp, x)


def _stage_mid(pp, dinv, wa, b, wb, relu, scale_out):
    w_in = pp.shape[2]
    d_mid = wa.shape[1] if wa is not None else w_in
    d_out = wb.shape[1] if wb is not None else d_mid

    def body(*refs):
        it = iter(refs)
        pp_ref = next(it)
        dinv_ref = next(it)
        wa_ref = next(it) if wa is not None else None
        b_ref = next(it)
        wb_ref = next(it) if wb is not None else None
        o_ref = next(it)
        dinv_b = dinv_ref[...]
        t = (pp_ref[0] + pp_ref[1]) * dinv_b
        if wa_ref is not None:
            t = jnp.dot(t, wa_ref[...], preferred_element_type=jnp.float32)
        t = t + b_ref[...][None, :]
        if relu:
            t = jnp.maximum(t, 0.0)
        if wb_ref is not None:
            t = jnp.dot(t, wb_ref[...], preferred_element_type=jnp.float32)
        if scale_out:
            t = t * dinv_b
        o_ref[...] = t

    in_specs = [
        pl.BlockSpec((2, BLK, w_in), lambda i: (0, i, 0)),
        pl.BlockSpec((BLK, 1), lambda i: (i, 0)),
    ]
    args = [pp, dinv]
    if wa is not None:
        in_specs.append(pl.BlockSpec(wa.shape, lambda i: (0, 0)))
        args.append(wa)
    in_specs.append(pl.BlockSpec((d_mid,), lambda i: (0,)))
    args.append(b)
    if wb is not None:
        in_specs.append(pl.BlockSpec(wb.shape, lambda i: (0, 0)))
        args.append(wb)

    return pl.pallas_call(
        body,
        grid=(GRID,),
        in_specs=in_specs,
        out_specs=pl.BlockSpec((BLK, d_out), lambda i: (i, 0)),
        out_shape=jax.ShapeDtypeStruct((N, d_out), jnp.float32),
    )(*args)


def kernel(x, edge_index, W0, b0, W1, b1, W2, b2, W3, b3, W4, b4,
           W5, b5, W6, b6, W7, b7, W8, b8, W9, b9):
    Ws = [W0, W1, W2, W3, W4, W5, W6, W7, W8, W9]
    bs = [b0, b1, b2, b3, b4, b5, b6, b7, b8, b9]

    loop = jnp.arange(N, dtype=jnp.int32)

    def edge_layout(c):
        _, nchunk, e_pad = _geom(128 if c == 56 else 16)
        padn = e_pad - E_TOT
        pad_iota = jnp.arange(padn, dtype=jnp.int32)
        src = jnp.concatenate([edge_index[0], loop, pad_iota % N])
        dst = jnp.concatenate([edge_index[1], loop, N + pad_iota % (N_ACC - N)])
        return src.reshape(NW, nchunk, c), dst.reshape(NW, nchunk, c)

    edges = {c: edge_layout(c) for c in sorted({_geom(w)[0] for w in (16, 128)})}

    ones16 = jnp.zeros((_geom(16)[0], 16), jnp.float32).at[:, 0].set(1.0)
    zeros = {w: jnp.zeros((ZROWS, w), jnp.float32) for w in (16, 32, 64, 128)}

    degp = _deg_call()(edges[_geom(16)[0]][1], ones16, zeros[16])
    dinv, g = _stage_a(degp, x)

    agg_first = [DIMS[i] <= DIMS[i + 1] for i in range(NUM_LAYERS)]

    for i in range(NUM_LAYERS):
        src3, dst3 = edges[_geom(g.shape[1])[0]]
        pp = _agg_call(g.shape[1])(g, src3, dst3, zeros[g.shape[1]])
        wa = Ws[i] if agg_first[i] else None
        if i < NUM_LAYERS - 1:
            wb = None if agg_first[i + 1] else Ws[i + 1]
            g = _stage_mid(pp, dinv, wa, bs[i], wb, relu=True, scale_out=True)
        else:
            g = _stage_mid(pp, dinv, wa, bs[i], None, relu=False, scale_out=False)
    return g


# speedup vs baseline: 31.0256x; 1.0335x over previous
import functools

import jax
import jax.numpy as jnp
from jax import lax
from jax.experimental import pallas as pl
from jax.experimental.pallas import tpu as pltpu
from jax.experimental.pallas import tpu_sc as plsc

N = 10000
E = 320000
E_TOT = E + N
NW = 32
NBUF = 4


def _geom(w):
    c = 56 if w == 128 else 128
    nc0 = -(-E_TOT // (NW * c))
    nchunk = nc0 + (-nc0 % NBUF)
    return c, nchunk, NW * c * nchunk
N_ACC = 10112
ZROWS = N_ACC // 16
DIMS = [128, 256, 128, 64, 32, 16, 32, 64, 128, 256, 128]
NUM_LAYERS = 10
BLK = 2528
GRID = -(-N // BLK)


def _sc_mesh():
    return plsc.VectorSubcoreMesh(core_axis_name="c", subcore_axis_name="s")


@functools.lru_cache(maxsize=None)
def _agg_call(w):
    C, NCHUNK, _ = _geom(w)

    @functools.partial(
        pl.kernel,
        mesh=_sc_mesh(),
        compiler_params=pltpu.CompilerParams(use_tc_tiling_on_sc=False),
        out_type=jax.ShapeDtypeStruct((2, N_ACC, w), jnp.float32),
        scratch_types=[
            pltpu.VMEM((NCHUNK, C), jnp.int32),
            pltpu.VMEM((NCHUNK, C), jnp.int32),
        ] + [pltpu.VMEM((C, w), jnp.float32) for _ in range(NBUF)] + [
            pltpu.VMEM_SHARED((N_ACC, w), jnp.float32),
        ] + [pltpu.SemaphoreType.DMA] * (2 * NBUF),
    )
    def agg(g_hbm, src_hbm, dst_hbm, zero_hbm, out_hbm,
            srcv, dstv, b0, b1, b2, b3, acc, g0, g1, g2, g3, s0, s1, s2, s3):
        bufs = (b0, b1, b2, b3)
        semg = (g0, g1, g2, g3)
        sems = (s0, s1, s2, s3)
        ci = lax.axis_index("c")
        si = lax.axis_index("s")
        wid = si * 2 + ci
        pltpu.sync_copy(zero_hbm, acc.at[pl.ds(si * ZROWS, ZROWS)])
        pltpu.sync_copy(src_hbm.at[wid], srcv)
        pltpu.sync_copy(dst_hbm.at[wid], dstv)
        plsc.subcore_barrier()

        def g_fire(j, s):
            pltpu.async_copy(g_hbm.at[srcv.at[j]], bufs[s], semg[s])

        def g_wait(s):
            pltpu.make_async_copy(g_hbm.at[srcv.at[0]], bufs[s], semg[s]).wait()

        def s_fire(j, s):
            pltpu.async_copy(bufs[s], acc.at[dstv.at[j]], sems[s], add=True)

        def s_wait(s):
            pltpu.make_async_copy(bufs[s], acc.at[dstv.at[0]], sems[s]).wait()

        g_fire(0, 0)
        g_fire(1, 1)
        g_fire(2, 2)
        g_wait(0)
        s_fire(0, 0)
        g_fire(3, 3)
        g_wait(1)
        s_fire(1, 1)
        s_wait(0)
        g_fire(4, 0)
        g_wait(2)
        s_fire(2, 2)
        s_wait(1)
        g_fire(5, 1)
        g_wait(3)
        s_fire(3, 3)

        def body(g, carry):
            j0 = 4 * g
            for s in range(NBUF):
                j = j0 + s
                s2 = (s + 2) % NBUF
                s_wait(s2)
                g_fire(jnp.minimum(j + 2, NCHUNK - 1), s2)
                g_wait(s)
                s_fire(j, s)
            return carry

        lax.fori_loop(1, NCHUNK // NBUF, body, 0)
        g_wait(0)
        g_wait(1)
        s_wait(2)
        s_wait(3)
        plsc.subcore_barrier()
        pltpu.sync_copy(acc.at[pl.ds(si * ZROWS, ZROWS)],
                        out_hbm.at[ci, pl.ds(si * ZROWS, ZROWS)])

    return agg


@functools.lru_cache(maxsize=None)
def _deg_call():
    C, NCHUNK, _ = _geom(16)

    @functools.partial(
        pl.kernel,
        mesh=_sc_mesh(),
        compiler_params=pltpu.CompilerParams(use_tc_tiling_on_sc=False),
        out_type=jax.ShapeDtypeStruct((2, N_ACC, 16), jnp.float32),
        scratch_types=[
            pltpu.VMEM((NCHUNK, C), jnp.int32),
            pltpu.VMEM((C, 16), jnp.float32),
            pltpu.VMEM_SHARED((N_ACC, 16), jnp.float32),
        ] + [pltpu.SemaphoreType.DMA] * NBUF,
    )
    def deg(dst_hbm, ones_hbm, zero_hbm, out_hbm, dstv, buf, acc, s0, s1, s2, s3):
        sems = (s0, s1, s2, s3)
        ci = lax.axis_index("c")
        si = lax.axis_index("s")
        wid = si * 2 + ci
        pltpu.sync_copy(zero_hbm, acc.at[pl.ds(si * ZROWS, ZROWS)])
        pltpu.sync_copy(ones_hbm, buf)
        pltpu.sync_copy(dst_hbm.at[wid], dstv)
        plsc.subcore_barrier()

        def s_fire(j, s):
            pltpu.async_copy(buf, acc.at[dstv.at[j]], sems[s], add=True)

        def s_wait(s):
            pltpu.make_async_copy(buf, acc.at[dstv.at[0]], sems[s]).wait()

        for s in range(NBUF):
            s_fire(s, s)

        def body(g, carry):
            j0 = NBUF * g
            for s in range(NBUF):
                s_wait(s)
                s_fire(j0 + s, s)
            return carry

        lax.fori_loop(1, NCHUNK // NBUF, body, 0)
        for s in range(NBUF):
            s_wait(s)
        plsc.subcore_barrier()
        pltpu.sync_copy(acc.at[pl.ds(si * ZROWS, ZROWS)],
                        out_hbm.at[ci, pl.ds(si * ZROWS, ZROWS)])

    return deg


def _stage_a(degp, x):
    def body(degp_ref, x_ref, dinv_ref, g_ref):
        deg = degp_ref[0, :, 0:1] + degp_ref[1, :, 0:1]
        dinv = jnp.where(deg > 0, lax.rsqrt(jnp.maximum(deg, 1e-12)), 0.0)
        dinv_ref[...] = dinv
        g_ref[...] = x_ref[...] * dinv

    return pl.pallas_call(
        body,
        grid=(GRID,),
        in_specs=[
            pl.BlockSpec((2, BLK, 16), lambda i: (0, i, 0)),
            pl.BlockSpec((BLK, DIMS[0]), lambda i: (i, 0)),
        ],
        out_specs=[
            pl.BlockSpec((BLK, 1), lambda i: (i, 0)),
            pl.BlockSpec((BLK, DIMS[0]), lambda i: (i, 0)),
        ],
        out_shape=[
            jax.ShapeDtypeStruct((N, 1), jnp.float32),
            jax.ShapeDtypeStruct((N, DIMS[0]), jnp.float32),
        ],
    )(degp, x)


def _stage_mid(pp, dinv, wa, b, wb, relu, scale_out):
    w_in = pp.shape[2]
    d_mid = wa.shape[1] if wa is not None else w_in
    d_out = wb.shape[1] if wb is not None else d_mid

    def body(*refs):
        it = iter(refs)
        pp_ref = next(it)
        dinv_ref = next(it)
        wa_ref = next(it) if wa is not None else None
        b_ref = next(it)
        wb_ref = next(it) if wb is not None else None
        o_ref = next(it)
        dinv_b = dinv_ref[...]
        t = (pp_ref[0] + pp_ref[1]) * dinv_b
        if wa_ref is not None:
            t = jnp.dot(t, wa_ref[...], preferred_element_type=jnp.float32)
        t = t + b_ref[...][None, :]
        if relu:
            t = jnp.maximum(t, 0.0)
        if wb_ref is not None:
            t = jnp.dot(t, wb_ref[...], preferred_element_type=jnp.float32)
        if scale_out:
            t = t * dinv_b
        o_ref[...] = t

    in_specs = [
        pl.BlockSpec((2, BLK, w_in), lambda i: (0, i, 0)),
        pl.BlockSpec((BLK, 1), lambda i: (i, 0)),
    ]
    args = [pp, dinv]
    if wa is not None:
        in_specs.append(pl.BlockSpec(wa.shape, lambda i: (0, 0)))
        args.append(wa)
    in_specs.append(pl.BlockSpec((d_mid,), lambda i: (0,)))
    args.append(b)
    if wb is not None:
        in_specs.append(pl.BlockSpec(wb.shape, lambda i: (0, 0)))
        args.append(wb)

    return pl.pallas_call(
        body,
        grid=(GRID,),
        in_specs=in_specs,
        out_specs=pl.BlockSpec((BLK, d_out), lambda i: (i, 0)),
        out_shape=jax.ShapeDtypeStruct((N, d_out), jnp.float32),
    )(*args)


def kernel(x, edge_index, W0, b0, W1, b1, W2, b2, W3, b3, W4, b4,
           W5, b5, W6, b6, W7, b7, W8, b8, W9, b9):
    Ws = [W0, W1, W2, W3, W4, W5, W6, W7, W8, W9]
    bs = [b0, b1, b2, b3, b4, b5, b6, b7, b8, b9]

    loop = jnp.arange(N, dtype=jnp.int32)

    def edge_layout(c):
        _, nchunk, e_pad = _geom(128 if c == 56 else 16)
        padn = e_pad - E_TOT
        pad_iota = jnp.arange(padn, dtype=jnp.int32)
        src = jnp.concatenate([edge_index[0], loop, pad_iota % N])
        dst = jnp.concatenate([edge_index[1], loop, N + pad_iota % (N_ACC - N)])
        return src.reshape(NW, nchunk, c), dst.reshape(NW, nchunk, c)

    edges = {c: edge_layout(c) for c in sorted({_geom(w)[0] for w in (16, 128)})}

    ones16 = jnp.zeros((_geom(16)[0], 16), jnp.float32).at[:, 0].set(1.0)
    zeros = {w: jnp.zeros((ZROWS, w), jnp.float32) for w in (16, 32, 64, 128)}

    degp = _deg_call()(edges[_geom(16)[0]][1], ones16, zeros[16])
    dinv, g = _stage_a(degp, x)

    agg_first = [DIMS[i] <= DIMS[i + 1] for i in range(NUM_LAYERS)]

    for i in range(NUM_LAYERS):
        src3, dst3 = edges[_geom(g.shape[1])[0]]
        pp = _agg_call(g.shape[1])(g, src3, dst3, zeros[g.shape[1]])
        wa = Ws[i] if agg_first[i] else None
        if i < NUM_LAYERS - 1:
            wb = None if agg_first[i + 1] else Ws[i + 1]
            g = _stage_mid(pp, dinv, wa, bs[i], wb, relu=True, scale_out=True)
        else:
            g = _stage_mid(pp, dinv, wa, bs[i], None, relu=False, scale_out=False)
    return g


# speedup vs baseline: 31.5805x; 1.0179x over previous
import functools

import jax
import jax.numpy as jnp
from jax import lax
from jax.experimental import pallas as pl
from jax.experimental.pallas import tpu as pltpu
from jax.experimental.pallas import tpu_sc as plsc

N = 10000
E = 320000
E_TOT = E + N
NW = 32
NBUF = 4


def _geom(w):
    c, b = (56, 1) if w == 128 else (128, 1)
    nc0 = -(-E_TOT // (NW * b * c))
    nchunk = nc0 + (-nc0 % NBUF)
    return c, b, nchunk, NW * b * c * nchunk
N_ACC = 10112
ZROWS = N_ACC // 16
DIMS = [128, 256, 128, 64, 32, 16, 32, 64, 128, 256, 128]
NUM_LAYERS = 10
BLK = 2528
GRID = -(-N // BLK)


def _sc_mesh():
    return plsc.VectorSubcoreMesh(core_axis_name="c", subcore_axis_name="s")


@functools.lru_cache(maxsize=None)
def _agg_call(w):
    C, B, NCHUNK, _ = _geom(w)

    @functools.partial(
        pl.kernel,
        mesh=_sc_mesh(),
        compiler_params=pltpu.CompilerParams(use_tc_tiling_on_sc=False),
        out_type=jax.ShapeDtypeStruct((2, N_ACC, w), jnp.float32),
        scratch_types=[
            pltpu.VMEM((NCHUNK, C) if B == 1 else (NCHUNK, B, C), jnp.int32),
            pltpu.VMEM((NCHUNK, C) if B == 1 else (NCHUNK, B, C), jnp.int32),
        ] + [pltpu.VMEM((C, w) if B == 1 else (B, C, w), jnp.float32)
             for _ in range(NBUF)] + [
            pltpu.VMEM_SHARED((N_ACC, w), jnp.float32),
        ] + [pltpu.SemaphoreType.DMA] * (2 * NBUF),
    )
    def agg(g_hbm, src_hbm, dst_hbm, zero_hbm, out_hbm,
            srcv, dstv, b0, b1, b2, b3, acc, g0, g1, g2, g3, s0, s1, s2, s3):
        bufs = (b0, b1, b2, b3)
        semg = (g0, g1, g2, g3)
        sems = (s0, s1, s2, s3)
        ci = lax.axis_index("c")
        si = lax.axis_index("s")
        wid = si * 2 + ci
        pltpu.async_copy(zero_hbm, acc.at[pl.ds(si * ZROWS, ZROWS)], g0)
        pltpu.async_copy(src_hbm.at[wid], srcv, g1)
        pltpu.async_copy(dst_hbm.at[wid], dstv, g2)
        pltpu.make_async_copy(zero_hbm, acc.at[pl.ds(si * ZROWS, ZROWS)], g0).wait()
        pltpu.make_async_copy(src_hbm.at[wid], srcv, g1).wait()
        pltpu.make_async_copy(dst_hbm.at[wid], dstv, g2).wait()
        plsc.subcore_barrier()

        def g_fire(j, s):
            pltpu.async_copy(g_hbm.at[srcv.at[j]], bufs[s], semg[s])

        def g_wait(s):
            pltpu.make_async_copy(g_hbm.at[srcv.at[0]], bufs[s], semg[s]).wait()

        def s_fire(j, s):
            pltpu.async_copy(bufs[s], acc.at[dstv.at[j]], sems[s], add=True)

        def s_wait(s):
            pltpu.make_async_copy(bufs[s], acc.at[dstv.at[0]], sems[s]).wait()

        g_fire(0, 0)
        g_fire(1, 1)
        g_fire(2, 2)
        g_wait(0)
        s_fire(0, 0)
        g_fire(3, 3)
        g_wait(1)
        s_fire(1, 1)
        s_wait(0)
        g_fire(4, 0)
        g_wait(2)
        s_fire(2, 2)
        s_wait(1)
        g_fire(5, 1)
        g_wait(3)
        s_fire(3, 3)

        def body(g, carry):
            j0 = 4 * g
            for s in range(NBUF):
                j = j0 + s
                s2 = (s + 2) % NBUF
                s_wait(s2)
                g_fire(jnp.minimum(j + 2, NCHUNK - 1), s2)
                g_wait(s)
                s_fire(j, s)
            return carry

        lax.fori_loop(1, NCHUNK // NBUF, body, 0)
        g_wait(0)
        g_wait(1)
        s_wait(2)
        s_wait(3)
        plsc.subcore_barrier()
        pltpu.sync_copy(acc.at[pl.ds(si * ZROWS, ZROWS)],
                        out_hbm.at[ci, pl.ds(si * ZROWS, ZROWS)])

    return agg


@functools.lru_cache(maxsize=None)
def _deg_call():
    C, B, NCHUNK, _ = _geom(16)

    @functools.partial(
        pl.kernel,
        mesh=_sc_mesh(),
        compiler_params=pltpu.CompilerParams(use_tc_tiling_on_sc=False),
        out_type=jax.ShapeDtypeStruct((2, N_ACC, 16), jnp.float32),
        scratch_types=[
            pltpu.VMEM((NCHUNK, C), jnp.int32),
            pltpu.VMEM((C, 16), jnp.float32),
            pltpu.VMEM_SHARED((N_ACC, 16), jnp.float32),
        ] + [pltpu.SemaphoreType.DMA] * NBUF,
    )
    def deg(dst_hbm, ones_hbm, zero_hbm, out_hbm, dstv, buf, acc, s0, s1, s2, s3):
        sems = (s0, s1, s2, s3)
        ci = lax.axis_index("c")
        si = lax.axis_index("s")
        wid = si * 2 + ci
        pltpu.sync_copy(zero_hbm, acc.at[pl.ds(si * ZROWS, ZROWS)])
        pltpu.sync_copy(ones_hbm, buf)
        pltpu.sync_copy(dst_hbm.at[wid], dstv)
        plsc.subcore_barrier()

        def s_fire(j, s):
            pltpu.async_copy(buf, acc.at[dstv.at[j]], sems[s], add=True)

        def s_wait(s):
            pltpu.make_async_copy(buf, acc.at[dstv.at[0]], sems[s]).wait()

        for s in range(NBUF):
            s_fire(s, s)

        def body(g, carry):
            j0 = NBUF * g
            for s in range(NBUF):
                s_wait(s)
                s_fire(j0 + s, s)
            return carry

        lax.fori_loop(1, NCHUNK // NBUF, body, 0)
        for s in range(NBUF):
            s_wait(s)
        plsc.subcore_barrier()
        pltpu.sync_copy(acc.at[pl.ds(si * ZROWS, ZROWS)],
                        out_hbm.at[ci, pl.ds(si * ZROWS, ZROWS)])

    return deg


def _stage_a(degp, x):
    def body(degp_ref, x_ref, dinv_ref, g_ref):
        deg = degp_ref[0, :, 0:1] + degp_ref[1, :, 0:1]
        dinv = jnp.where(deg > 0, lax.rsqrt(jnp.maximum(deg, 1e-12)), 0.0)
        dinv_ref[...] = dinv
        g_ref[...] = x_ref[...] * dinv

    return pl.pallas_call(
        body,
        grid=(GRID,),
        in_specs=[
            pl.BlockSpec((2, BLK, 16), lambda i: (0, i, 0)),
            pl.BlockSpec((BLK, DIMS[0]), lambda i: (i, 0)),
        ],
        out_specs=[
            pl.BlockSpec((BLK, 1), lambda i: (i, 0)),
            pl.BlockSpec((BLK, DIMS[0]), lambda i: (i, 0)),
        ],
        out_shape=[
            jax.ShapeDtypeStruct((N, 1), jnp.float32),
            jax.ShapeDtypeStruct((N, DIMS[0]), jnp.float32),
        ],
    )(degp, x)


def _stage_mid(pp, dinv, wa, b, wb, relu, scale_out):
    w_in = pp.shape[2]
    d_mid = wa.shape[1] if wa is not None else w_in
    d_out = wb.shape[1] if wb is not None else d_mid

    def body(*refs):
        it = iter(refs)
        pp_ref = next(it)
        dinv_ref = next(it)
        wa_ref = next(it) if wa is not None else None
        b_ref = next(it)
        wb_ref = next(it) if wb is not None else None
        o_ref = next(it)
        dinv_b = dinv_ref[...]
        t = (pp_ref[0] + pp_ref[1]) * dinv_b
        if wa_ref is not None:
            t = jnp.dot(t, wa_ref[...], preferred_element_type=jnp.float32)
        t = t + b_ref[...][None, :]
        if relu:
            t = jnp.maximum(t, 0.0)
        if wb_ref is not None:
            t = jnp.dot(t, wb_ref[...], preferred_element_type=jnp.float32)
        if scale_out:
            t = t * dinv_b
        o_ref[...] = t

    in_specs = [
        pl.BlockSpec((2, BLK, w_in), lambda i: (0, i, 0)),
        pl.BlockSpec((BLK, 1), lambda i: (i, 0)),
    ]
    args = [pp, dinv]
    if wa is not None:
        in_specs.append(pl.BlockSpec(wa.shape, lambda i: (0, 0)))
        args.append(wa)
    in_specs.append(pl.BlockSpec((d_mid,), lambda i: (0,)))
    args.append(b)
    if wb is not None:
        in_specs.append(pl.BlockSpec(wb.shape, lambda i: (0, 0)))
        args.append(wb)

    return pl.pallas_call(
        body,
        grid=(GRID,),
        in_specs=in_specs,
        out_specs=pl.BlockSpec((BLK, d_out), lambda i: (i, 0)),
        out_shape=jax.ShapeDtypeStruct((N, d_out), jnp.float32),
    )(*args)


def kernel(x, edge_index, W0, b0, W1, b1, W2, b2, W3, b3, W4, b4,
           W5, b5, W6, b6, W7, b7, W8, b8, W9, b9):
    Ws = [W0, W1, W2, W3, W4, W5, W6, W7, W8, W9]
    bs = [b0, b1, b2, b3, b4, b5, b6, b7, b8, b9]

    loop = jnp.arange(N, dtype=jnp.int32)

    def edge_layout(w):
        c, b, nchunk, e_pad = _geom(w)
        padn = e_pad - E_TOT
        pad_iota = jnp.arange(padn, dtype=jnp.int32)
        src = jnp.concatenate([edge_index[0], loop, pad_iota % N])
        dst = jnp.concatenate([edge_index[1], loop, N + pad_iota % (N_ACC - N)])
        shape = (NW, nchunk, c) if b == 1 else (NW, nchunk, b, c)
        return src.reshape(shape), dst.reshape(shape)

    edges = {_geom(w)[:2]: edge_layout(w) for w in (16, 128)}

    def edges_for(w):
        return edges[_geom(w)[:2]]

    ones16 = jnp.zeros((_geom(16)[0], 16), jnp.float32).at[:, 0].set(1.0)
    zeros = {w: jnp.zeros((ZROWS, w), jnp.float32) for w in (16, 32, 64, 128)}

    degp = _deg_call()(edges_for(16)[1], ones16, zeros[16])
    dinv, g = _stage_a(degp, x)

    agg_first = [DIMS[i] <= DIMS[i + 1] for i in range(NUM_LAYERS)]

    for i in range(NUM_LAYERS):
        src3, dst3 = edges_for(g.shape[1])
        pp = _agg_call(g.shape[1])(g, src3, dst3, zeros[g.shape[1]])
        wa = Ws[i] if agg_first[i] else None
        if i < NUM_LAYERS - 1:
            wb = None if agg_first[i + 1] else Ws[i + 1]
            g = _stage_mid(pp, dinv, wa, bs[i], wb, relu=True, scale_out=True)
        else:
            g = _stage_mid(pp, dinv, wa, bs[i], None, relu=False, scale_out=False)
    return g
